# baseline (speedup 1.0000x reference)
import functools

import jax
import jax.numpy as jnp
from jax import lax
from jax.experimental import pallas as pl
from jax.experimental.pallas import tpu as pltpu
from jax.experimental.pallas import tpu_sc as plsc

N_ROWS = 128
ROW_LEN = 32768
K = 10
LANES = 16
BLOCK_VREGS = 16
BLOCK_ELEMS = BLOCK_VREGS * LANES
N_BLOCKS = ROW_LEN // BLOCK_ELEMS

N_CORES = 2
N_SUBCORES = 16
ROWS_PER_WORKER = N_ROWS // (N_CORES * N_SUBCORES)

_NEG = float(-jnp.inf)
_POS = float(jnp.inf)


def _iota():
    return lax.iota(jnp.int32, LANES)


def _insert_desc(regs, v):
    out = []
    c = v
    for r in regs:
        out.append(jnp.maximum(r, c))
        c = jnp.minimum(r, c)
    return tuple(out)


def _insert_asc(regs, v):
    out = []
    c = v
    for r in regs:
        out.append(jnp.minimum(r, c))
        c = jnp.maximum(r, c)
    return tuple(out)


def _bitonic_16(regs, desc):
    regs = list(regs)
    for d in (8, 4, 2, 1):
        for k in range(16):
            if k & d:
                continue
            hi = jnp.maximum(regs[k], regs[k + d])
            lo = jnp.minimum(regs[k], regs[k + d])
            regs[k] = hi if desc else lo
            regs[k + d] = lo if desc else hi
    return regs


def _merge_tree(regs, desc):
    regs = list(regs)
    for dist in (1, 2, 4, 8):
        idx = _iota() ^ dist
        partner = [r[idx] for r in regs]
        n = len(regs)
        merged = []
        for k in range(16):
            a = regs[k] if k < n else None
            b = partner[15 - k] if 15 - k < n else None
            if a is None:
                merged.append(b)
            elif b is None:
                merged.append(a)
            else:
                merged.append(jnp.maximum(a, b) if desc else jnp.minimum(a, b))
        regs = _bitonic_16(merged, desc)
    return regs


K2 = 12


def _sort4(v, desc):
    v = list(v)
    for a, b in [(0, 1), (2, 3), (0, 2), (1, 3), (1, 2)]:
        hi = jnp.maximum(v[a], v[b])
        lo = jnp.minimum(v[a], v[b])
        v[a], v[b] = (hi, lo) if desc else (lo, hi)
    return v


def _merge_12_4(L, S, desc):
    x = list(L) + [S[3], S[2], S[1], S[0]]

    def cx(i, j):
        hi = jnp.maximum(x[i], x[j])
        lo = jnp.minimum(x[i], x[j])
        x[i], x[j] = (hi, lo) if desc else (lo, hi)

    for k in range(8):
        cx(k, k + 8)
    for d in (4, 2, 1):
        for k in range(8):
            if not k & d:
                cx(k, k + d)
    for k in range(8, 12):
        cx(k, k + 4)
    for d in (2, 1):
        for k in range(8, 12):
            if not (k - 8) & d:
                cx(k, k + d)
    return x[:12]


def _bfly_max(v):
    for d in (1, 2, 4, 8):
        v = jnp.maximum(v, v[_iota() ^ d])
    return v


def _bfly_min(v):
    for d in (1, 2, 4, 8):
        v = jnp.minimum(v, v[_iota() ^ d])
    return v


def _assemble(regs):
    iota = _iota()
    acc = regs[0]
    for k in range(1, K):
        acc = jnp.where(iota == k, regs[k], acc)
    return acc


def _body(x_hbm, out_hbm, row_v, bm_v, bn_v, st_v, out_v, sem0, sem1):
    wid = lax.axis_index("s") * N_CORES + lax.axis_index("c")

    neg = jnp.full((LANES,), _NEG, jnp.float32)
    pos = jnp.full((LANES,), _POS, jnp.float32)

    row0 = wid * ROWS_PER_WORKER

    def buf(parity):
        return row_v.at[pl.ds(parity * ROW_LEN, ROW_LEN)]

    pltpu.async_copy(x_hbm.at[row0], buf(0), sem0)
    pltpu.async_copy(x_hbm.at[row0 + 1], buf(1), sem1)

    def row_work(t, carry):
        row = row0 + t
        even = t % 2 == 0

        @pl.when(even)
        def _():
            pltpu.make_async_copy(x_hbm.at[row], buf(0), sem0).wait()

        @pl.when(jnp.logical_not(even))
        def _():
            pltpu.make_async_copy(x_hbm.at[row], buf(1), sem1).wait()

        cur = (t % 2) * ROW_LEN

        @plsc.parallel_loop(
            0, N_BLOCKS, unroll=2, carry=(neg,) * K + (pos,) * K
        )
        def regs(b, regs):
            rs, ss = regs[:K], regs[K:]
            base = cur + b * BLOCK_ELEMS
            vs = [
                row_v[pl.ds(base + j * LANES, LANES)]
                for j in range(BLOCK_VREGS)
            ]
            bm = vs[0]
            bn = vs[0]
            for v in vs[1:]:
                bm = jnp.maximum(bm, v)
                bn = jnp.minimum(bn, v)
            bm_v[pl.ds(b * LANES, LANES)] = bm
            bn_v[pl.ds(b * LANES, LANES)] = bn
            return _insert_desc(rs, bm) + _insert_asc(ss, bn)

        theta_t = _merge_tree(regs[:K], True)[K - 1][0]
        theta_b = _merge_tree(regs[K:], False)[K - 1][0]

        for i in range(K):
            st_v[pl.ds(i * LANES, LANES)] = neg
            st_v[pl.ds((K + i) * LANES, LANES)] = pos

        theta_t_v = jnp.full((LANES,), 1.0, jnp.float32) * theta_t
        theta_b_v = jnp.full((LANES,), 1.0, jnp.float32) * theta_b
        zero = jnp.zeros((LANES,), jnp.float32)
        st_v[pl.ds(2 * K * LANES, LANES)] = zero

        def rescan(b, top):
            cap1 = neg if top else pos
            cap2 = neg if top else pos
            cnt = zero
            for j in range(BLOCK_VREGS):
                v = row_v[pl.ds(cur + b * BLOCK_ELEMS + j * LANES, LANES)]
                if top:
                    m = v >= theta_t_v
                    w = jnp.where(m, v, neg)
                    hi = jnp.maximum(cap1, w)
                    lo = jnp.minimum(cap1, w)
                    cap1 = hi
                    cap2 = jnp.maximum(cap2, lo)
                else:
                    m = v <= theta_b_v
                    w = jnp.where(m, v, pos)
                    lo = jnp.minimum(cap1, w)
                    hi = jnp.maximum(cap1, w)
                    cap1 = lo
                    cap2 = jnp.minimum(cap2, hi)
                cnt = cnt + jnp.where(m, 1.0, 0.0)
            g = st_v[pl.ds(2 * K * LANES, LANES)]
            st_v[pl.ds(2 * K * LANES, LANES)] = jnp.maximum(g, cnt)
            off = 0 if top else K * LANES
            regs = tuple(
                st_v[pl.ds(off + i * LANES, LANES)] for i in range(K)
            )
            if top:
                regs = _insert_desc(_insert_desc(regs, cap1), cap2)
            else:
                regs = _insert_asc(_insert_asc(regs, cap1), cap2)
            for i in range(K):
                st_v[pl.ds(off + i * LANES, LANES)] = regs[i]

        CGROUP = 8

        def c_group(gi, c):
            b0 = gi * CGROUP
            bms = [
                bm_v[pl.ds((b0 + q) * LANES, LANES)] for q in range(CGROUP)
            ]
            bns = [
                bn_v[pl.ds((b0 + q) * LANES, LANES)] for q in range(CGROUP)
            ]
            tts = [bm - theta_t_v for bm in bms]
            tbs = [theta_b_v - bn for bn in bns]
            scomb = [
                _bfly_max(jnp.maximum(tt, tb)) for tt, tb in zip(tts, tbs)
            ]
            for q in range(CGROUP):
                @pl.when(scomb[q][0] >= 0.0)
                def _(q=q):
                    st = _bfly_max(tts[q])
                    sb = _bfly_max(tbs[q])

                    @pl.when(st[0] >= 0.0)
                    def _(b=b0 + q):
                        rescan(b, True)

                    @pl.when(sb[0] >= 0.0)
                    def _(b=b0 + q):
                        rescan(b, False)

            return c

        lax.fori_loop(0, N_BLOCKS // CGROUP, c_group, jnp.int32(0))

        g = st_v[pl.ds(2 * K * LANES, LANES)]
        s_g = _bfly_max(g)[0]

        @pl.when(s_g >= 3.0)
        def _():
            for i in range(K):
                st_v[pl.ds(i * LANES, LANES)] = neg
                st_v[pl.ds((K + i) * LANES, LANES)] = pos

            def c_body(b, c):
                bm = bm_v[pl.ds(b * LANES, LANES)]
                bn = bn_v[pl.ds(b * LANES, LANES)]
                s_mx = _bfly_max(bm)[0]
                s_mn = _bfly_min(bn)[0]

                @pl.when(s_mx >= theta_t)
                def _():
                    rs = tuple(
                        st_v[pl.ds(i * LANES, LANES)] for i in range(K)
                    )
                    for j in range(BLOCK_VREGS):
                        v = row_v[
                            pl.ds(cur + b * BLOCK_ELEMS + j * LANES, LANES)
                        ]
                        rs = _insert_desc(rs, v)
                    for i in range(K):
                        st_v[pl.ds(i * LANES, LANES)] = rs[i]

                @pl.when(s_mn <= theta_b)
                def _():
                    ss = tuple(
                        st_v[pl.ds((K + i) * LANES, LANES)] for i in range(K)
                    )
                    for j in range(BLOCK_VREGS):
                        v = row_v[
                            pl.ds(cur + b * BLOCK_ELEMS + j * LANES, LANES)
                        ]
                        ss = _insert_asc(ss, v)
                    for i in range(K):
                        st_v[pl.ds((K + i) * LANES, LANES)] = ss[i]

                return c

            lax.fori_loop(0, N_BLOCKS, c_body, jnp.int32(0))

        @pl.when(jnp.logical_and(even, t + 2 < ROWS_PER_WORKER))
        def _():
            pltpu.async_copy(x_hbm.at[row + 2], buf(0), sem0)

        @pl.when(jnp.logical_and(jnp.logical_not(even),
                                 t + 2 < ROWS_PER_WORKER))
        def _():
            pltpu.async_copy(x_hbm.at[row + 2], buf(1), sem1)

        rs = tuple(st_v[pl.ds(i * LANES, LANES)] for i in range(K))
        ss = tuple(st_v[pl.ds((K + i) * LANES, LANES)] for i in range(K))
        out_v[pl.ds(t * 2 * LANES, LANES)] = _assemble(_merge_tree(rs, True))
        out_v[pl.ds(t * 2 * LANES + LANES, LANES)] = _assemble(
            _merge_tree(ss, False)
        )
        return carry

    lax.fori_loop(0, ROWS_PER_WORKER, row_work, jnp.int32(0))

    out_len = ROWS_PER_WORKER * 2 * LANES
    pltpu.sync_copy(out_v, out_hbm.at[pl.ds(wid * out_len, out_len)])


@functools.cache
def _get_sc_extreme():
    return pl.kernel(
        _body,
        out_type=jax.ShapeDtypeStruct((N_ROWS * 2 * LANES,), jnp.float32),
        mesh=plsc.VectorSubcoreMesh(
            core_axis_name="c",
            subcore_axis_name="s",
            num_cores=N_CORES,
            num_subcores=N_SUBCORES,
        ),
        scratch_types=[
            pltpu.VMEM((2 * ROW_LEN,), jnp.float32),
            pltpu.VMEM((N_BLOCKS * LANES,), jnp.float32),
            pltpu.VMEM((N_BLOCKS * LANES,), jnp.float32),
            pltpu.VMEM(((2 * K + 1) * LANES,), jnp.float32),
            pltpu.VMEM((ROWS_PER_WORKER * 2 * LANES,), jnp.float32),
            pltpu.SemaphoreType.DMA,
            pltpu.SemaphoreType.DMA,
        ],
    )


@jax.jit
def kernel(x):
    padded = _get_sc_extreme()(x).reshape(N_ROWS, 2 * LANES)
    return jnp.concatenate([padded[:, :K], padded[:, LANES:LANES + K]], axis=1)


# speedup vs baseline: 1.0779x; 1.0779x over previous
import functools

import jax
import jax.numpy as jnp
from jax import lax
from jax.experimental import pallas as pl
from jax.experimental.pallas import tpu as pltpu
from jax.experimental.pallas import tpu_sc as plsc

N_ROWS = 128
ROW_LEN = 32768
K = 10
LANES = 16
BLOCK_VREGS = 16
BLOCK_ELEMS = BLOCK_VREGS * LANES
N_BLOCKS = ROW_LEN // BLOCK_ELEMS

N_CORES = 2
N_SUBCORES = 16
ROWS_PER_WORKER = N_ROWS // (N_CORES * N_SUBCORES)

_NEG = float(-jnp.inf)
_POS = float(jnp.inf)


def _iota():
    return lax.iota(jnp.int32, LANES)


def _insert_desc(regs, v):
    out = []
    c = v
    for r in regs:
        out.append(jnp.maximum(r, c))
        c = jnp.minimum(r, c)
    return tuple(out)


def _insert_asc(regs, v):
    out = []
    c = v
    for r in regs:
        out.append(jnp.minimum(r, c))
        c = jnp.maximum(r, c)
    return tuple(out)


def _bitonic_16(regs, desc):
    regs = list(regs)
    for d in (8, 4, 2, 1):
        for k in range(16):
            if k & d:
                continue
            hi = jnp.maximum(regs[k], regs[k + d])
            lo = jnp.minimum(regs[k], regs[k + d])
            regs[k] = hi if desc else lo
            regs[k + d] = lo if desc else hi
    return regs


def _merge_tree(regs, desc):
    regs = list(regs)
    for dist in (1, 2, 4, 8):
        idx = _iota() ^ dist
        partner = [r[idx] for r in regs]
        n = len(regs)
        merged = []
        for k in range(16):
            a = regs[k] if k < n else None
            b = partner[15 - k] if 15 - k < n else None
            if a is None:
                merged.append(b)
            elif b is None:
                merged.append(a)
            else:
                merged.append(jnp.maximum(a, b) if desc else jnp.minimum(a, b))
        regs = _bitonic_16(merged, desc)
    return regs


K2 = 12


def _sort4(v, desc):
    v = list(v)
    for a, b in [(0, 1), (2, 3), (0, 2), (1, 3), (1, 2)]:
        hi = jnp.maximum(v[a], v[b])
        lo = jnp.minimum(v[a], v[b])
        v[a], v[b] = (hi, lo) if desc else (lo, hi)
    return v


def _merge_12_4(L, S, desc):
    x = list(L) + [S[3], S[2], S[1], S[0]]

    def cx(i, j):
        hi = jnp.maximum(x[i], x[j])
        lo = jnp.minimum(x[i], x[j])
        x[i], x[j] = (hi, lo) if desc else (lo, hi)

    for k in range(8):
        cx(k, k + 8)
    for d in (4, 2, 1):
        for k in range(8):
            if not k & d:
                cx(k, k + d)
    for k in range(8, 12):
        cx(k, k + 4)
    for d in (2, 1):
        for k in range(8, 12):
            if not (k - 8) & d:
                cx(k, k + d)
    return x[:12]


def _bfly_max(v):
    for d in (1, 2, 4, 8):
        v = jnp.maximum(v, v[_iota() ^ d])
    return v


def _bfly_min(v):
    for d in (1, 2, 4, 8):
        v = jnp.minimum(v, v[_iota() ^ d])
    return v


def _assemble(regs):
    iota = _iota()
    acc = regs[0]
    for k in range(1, K):
        acc = jnp.where(iota == k, regs[k], acc)
    return acc


def _body(x_hbm, out_hbm, row_v, bm_v, bn_v, st_v, out_v, sem0, sem1):
    wid = lax.axis_index("s") * N_CORES + lax.axis_index("c")

    neg = jnp.full((LANES,), _NEG, jnp.float32)
    pos = jnp.full((LANES,), _POS, jnp.float32)

    row0 = wid * ROWS_PER_WORKER

    def buf(parity):
        return row_v.at[pl.ds(parity * ROW_LEN, ROW_LEN)]

    pltpu.async_copy(x_hbm.at[row0], buf(0), sem0)
    pltpu.async_copy(x_hbm.at[row0 + 1], buf(1), sem1)

    def row_work(t, carry):
        row = row0 + t
        even = t % 2 == 0

        @pl.when(even)
        def _():
            pltpu.make_async_copy(x_hbm.at[row], buf(0), sem0).wait()

        @pl.when(jnp.logical_not(even))
        def _():
            pltpu.make_async_copy(x_hbm.at[row], buf(1), sem1).wait()

        cur = (t % 2) * ROW_LEN

        @plsc.parallel_loop(
            0, N_BLOCKS, unroll=2, carry=(neg,) * K + (pos,) * K
        )
        def regs(b, regs):
            rs, ss = regs[:K], regs[K:]
            base = cur + b * BLOCK_ELEMS
            vs = [
                row_v[pl.ds(base + j * LANES, LANES)]
                for j in range(BLOCK_VREGS)
            ]
            bm = vs[0]
            bn = vs[0]
            for v in vs[1:]:
                bm = jnp.maximum(bm, v)
                bn = jnp.minimum(bn, v)
            bm_v[pl.ds(b * LANES, LANES)] = bm
            bn_v[pl.ds(b * LANES, LANES)] = bn
            return _insert_desc(rs, bm) + _insert_asc(ss, bn)

        theta_t = _merge_tree(regs[:K], True)[K - 1][0]
        theta_b = _merge_tree(regs[K:], False)[K - 1][0]

        for i in range(K):
            st_v[pl.ds(i * LANES, LANES)] = neg
            st_v[pl.ds((K + i) * LANES, LANES)] = pos

        theta_t_v = jnp.full((LANES,), 1.0, jnp.float32) * theta_t
        theta_b_v = jnp.full((LANES,), 1.0, jnp.float32) * theta_b
        zero = jnp.zeros((LANES,), jnp.float32)
        st_v[pl.ds(2 * K * LANES, LANES)] = zero

        def rescan(b, top):
            cap1 = neg if top else pos
            cap2 = neg if top else pos
            cnt = zero
            for j in range(BLOCK_VREGS):
                v = row_v[pl.ds(cur + b * BLOCK_ELEMS + j * LANES, LANES)]
                if top:
                    m = v >= theta_t_v
                    w = jnp.where(m, v, neg)
                    hi = jnp.maximum(cap1, w)
                    lo = jnp.minimum(cap1, w)
                    cap1 = hi
                    cap2 = jnp.maximum(cap2, lo)
                else:
                    m = v <= theta_b_v
                    w = jnp.where(m, v, pos)
                    lo = jnp.minimum(cap1, w)
                    hi = jnp.maximum(cap1, w)
                    cap1 = lo
                    cap2 = jnp.minimum(cap2, hi)
                cnt = cnt + jnp.where(m, 1.0, 0.0)
            g = st_v[pl.ds(2 * K * LANES, LANES)]
            st_v[pl.ds(2 * K * LANES, LANES)] = jnp.maximum(g, cnt)
            off = 0 if top else K * LANES
            regs = tuple(
                st_v[pl.ds(off + i * LANES, LANES)] for i in range(K)
            )
            if top:
                regs = _insert_desc(_insert_desc(regs, cap1), cap2)
            else:
                regs = _insert_asc(_insert_asc(regs, cap1), cap2)
            for i in range(K):
                st_v[pl.ds(off + i * LANES, LANES)] = regs[i]

        CGROUP = 2

        def c_group(gi, c):
            b0 = gi * CGROUP
            bms = [
                bm_v[pl.ds((b0 + q) * LANES, LANES)] for q in range(CGROUP)
            ]
            bns = [
                bn_v[pl.ds((b0 + q) * LANES, LANES)] for q in range(CGROUP)
            ]
            tts = [bm - theta_t_v for bm in bms]
            tbs = [theta_b_v - bn for bn in bns]
            scomb = [
                _bfly_max(jnp.maximum(tt, tb)) for tt, tb in zip(tts, tbs)
            ]
            for q in range(CGROUP):
                @pl.when(scomb[q][0] >= 0.0)
                def _(q=q):
                    st = _bfly_max(tts[q])
                    sb = _bfly_max(tbs[q])

                    @pl.when(st[0] >= 0.0)
                    def _(b=b0 + q):
                        rescan(b, True)

                    @pl.when(sb[0] >= 0.0)
                    def _(b=b0 + q):
                        rescan(b, False)

            return c

        lax.fori_loop(0, N_BLOCKS // CGROUP, c_group, jnp.int32(0))

        g = st_v[pl.ds(2 * K * LANES, LANES)]
        s_g = _bfly_max(g)[0]

        @pl.when(s_g >= 3.0)
        def _():
            for i in range(K):
                st_v[pl.ds(i * LANES, LANES)] = neg
                st_v[pl.ds((K + i) * LANES, LANES)] = pos

            def c_body(b, c):
                bm = bm_v[pl.ds(b * LANES, LANES)]
                bn = bn_v[pl.ds(b * LANES, LANES)]
                s_mx = _bfly_max(bm)[0]
                s_mn = _bfly_min(bn)[0]

                @pl.when(s_mx >= theta_t)
                def _():
                    rs = tuple(
                        st_v[pl.ds(i * LANES, LANES)] for i in range(K)
                    )
                    for j in range(BLOCK_VREGS):
                        v = row_v[
                            pl.ds(cur + b * BLOCK_ELEMS + j * LANES, LANES)
                        ]
                        rs = _insert_desc(rs, v)
                    for i in range(K):
                        st_v[pl.ds(i * LANES, LANES)] = rs[i]

                @pl.when(s_mn <= theta_b)
                def _():
                    ss = tuple(
                        st_v[pl.ds((K + i) * LANES, LANES)] for i in range(K)
                    )
                    for j in range(BLOCK_VREGS):
                        v = row_v[
                            pl.ds(cur + b * BLOCK_ELEMS + j * LANES, LANES)
                        ]
                        ss = _insert_asc(ss, v)
                    for i in range(K):
                        st_v[pl.ds((K + i) * LANES, LANES)] = ss[i]

                return c

            lax.fori_loop(0, N_BLOCKS, c_body, jnp.int32(0))

        @pl.when(jnp.logical_and(even, t + 2 < ROWS_PER_WORKER))
        def _():
            pltpu.async_copy(x_hbm.at[row + 2], buf(0), sem0)

        @pl.when(jnp.logical_and(jnp.logical_not(even),
                                 t + 2 < ROWS_PER_WORKER))
        def _():
            pltpu.async_copy(x_hbm.at[row + 2], buf(1), sem1)

        rs = tuple(st_v[pl.ds(i * LANES, LANES)] for i in range(K))
        ss = tuple(st_v[pl.ds((K + i) * LANES, LANES)] for i in range(K))
        out_v[pl.ds(t * 2 * LANES, LANES)] = _assemble(_merge_tree(rs, True))
        out_v[pl.ds(t * 2 * LANES + LANES, LANES)] = _assemble(
            _merge_tree(ss, False)
        )
        return carry

    lax.fori_loop(0, ROWS_PER_WORKER, row_work, jnp.int32(0))

    out_len = ROWS_PER_WORKER * 2 * LANES
    pltpu.sync_copy(out_v, out_hbm.at[pl.ds(wid * out_len, out_len)])


@functools.cache
def _get_sc_extreme():
    return pl.kernel(
        _body,
        out_type=jax.ShapeDtypeStruct((N_ROWS * 2 * LANES,), jnp.float32),
        mesh=plsc.VectorSubcoreMesh(
            core_axis_name="c",
            subcore_axis_name="s",
            num_cores=N_CORES,
            num_subcores=N_SUBCORES,
        ),
        scratch_types=[
            pltpu.VMEM((2 * ROW_LEN,), jnp.float32),
            pltpu.VMEM((N_BLOCKS * LANES,), jnp.float32),
            pltpu.VMEM((N_BLOCKS * LANES,), jnp.float32),
            pltpu.VMEM(((2 * K + 1) * LANES,), jnp.float32),
            pltpu.VMEM((ROWS_PER_WORKER * 2 * LANES,), jnp.float32),
            pltpu.SemaphoreType.DMA,
            pltpu.SemaphoreType.DMA,
        ],
    )


@jax.jit
def kernel(x):
    padded = _get_sc_extreme()(x).reshape(N_ROWS, 2 * LANES)
    return jnp.concatenate([padded[:, :K], padded[:, LANES:LANES + K]], axis=1)


# speedup vs baseline: 1.0814x; 1.0032x over previous
import functools

import jax
import jax.numpy as jnp
from jax import lax
from jax.experimental import pallas as pl
from jax.experimental.pallas import tpu as pltpu
from jax.experimental.pallas import tpu_sc as plsc

N_ROWS = 128
ROW_LEN = 32768
K = 10
LANES = 16
BLOCK_VREGS = 16
BLOCK_ELEMS = BLOCK_VREGS * LANES
N_BLOCKS = ROW_LEN // BLOCK_ELEMS

N_CORES = 2
N_SUBCORES = 16
ROWS_PER_WORKER = N_ROWS // (N_CORES * N_SUBCORES)

_NEG = float(-jnp.inf)
_POS = float(jnp.inf)


def _iota():
    return lax.iota(jnp.int32, LANES)


def _insert_desc(regs, v):
    out = []
    c = v
    for r in regs:
        out.append(jnp.maximum(r, c))
        c = jnp.minimum(r, c)
    return tuple(out)


def _insert_asc(regs, v):
    out = []
    c = v
    for r in regs:
        out.append(jnp.minimum(r, c))
        c = jnp.maximum(r, c)
    return tuple(out)


def _bitonic_16(regs, desc):
    regs = list(regs)
    for d in (8, 4, 2, 1):
        for k in range(16):
            if k & d:
                continue
            hi = jnp.maximum(regs[k], regs[k + d])
            lo = jnp.minimum(regs[k], regs[k + d])
            regs[k] = hi if desc else lo
            regs[k + d] = lo if desc else hi
    return regs


def _merge_tree(regs, desc):
    regs = list(regs)
    for dist in (1, 2, 4, 8):
        idx = _iota() ^ dist
        partner = [r[idx] for r in regs]
        n = len(regs)
        merged = []
        for k in range(16):
            a = regs[k] if k < n else None
            b = partner[15 - k] if 15 - k < n else None
            if a is None:
                merged.append(b)
            elif b is None:
                merged.append(a)
            else:
                merged.append(jnp.maximum(a, b) if desc else jnp.minimum(a, b))
        regs = _bitonic_16(merged, desc)
    return regs


K2 = 12


def _sort4(v, desc):
    v = list(v)
    for a, b in [(0, 1), (2, 3), (0, 2), (1, 3), (1, 2)]:
        hi = jnp.maximum(v[a], v[b])
        lo = jnp.minimum(v[a], v[b])
        v[a], v[b] = (hi, lo) if desc else (lo, hi)
    return v


def _merge_12_4(L, S, desc):
    x = list(L) + [S[3], S[2], S[1], S[0]]

    def cx(i, j):
        hi = jnp.maximum(x[i], x[j])
        lo = jnp.minimum(x[i], x[j])
        x[i], x[j] = (hi, lo) if desc else (lo, hi)

    for k in range(8):
        cx(k, k + 8)
    for d in (4, 2, 1):
        for k in range(8):
            if not k & d:
                cx(k, k + d)
    for k in range(8, 12):
        cx(k, k + 4)
    for d in (2, 1):
        for k in range(8, 12):
            if not (k - 8) & d:
                cx(k, k + d)
    return x[:12]


def _bfly_max(v):
    for d in (1, 2, 4, 8):
        v = jnp.maximum(v, v[_iota() ^ d])
    return v


def _bfly_min(v):
    for d in (1, 2, 4, 8):
        v = jnp.minimum(v, v[_iota() ^ d])
    return v


def _assemble(regs):
    iota = _iota()
    acc = regs[0]
    for k in range(1, K):
        acc = jnp.where(iota == k, regs[k], acc)
    return acc


def _body(x_hbm, out_hbm, row_v, bm_v, bn_v, st_v, out_v, sem0, sem1):
    wid = lax.axis_index("s") * N_CORES + lax.axis_index("c")

    neg = jnp.full((LANES,), _NEG, jnp.float32)
    pos = jnp.full((LANES,), _POS, jnp.float32)

    row0 = wid * ROWS_PER_WORKER

    def buf(parity):
        return row_v.at[pl.ds(parity * ROW_LEN, ROW_LEN)]

    pltpu.async_copy(x_hbm.at[row0], buf(0), sem0)
    pltpu.async_copy(x_hbm.at[row0 + 1], buf(1), sem1)

    def row_work(t, carry):
        row = row0 + t
        even = t % 2 == 0

        @pl.when(even)
        def _():
            pltpu.make_async_copy(x_hbm.at[row], buf(0), sem0).wait()

        @pl.when(jnp.logical_not(even))
        def _():
            pltpu.make_async_copy(x_hbm.at[row], buf(1), sem1).wait()

        cur = (t % 2) * ROW_LEN

        @plsc.parallel_loop(
            0, N_BLOCKS, unroll=2, carry=(neg,) * K + (pos,) * K
        )
        def regs(b, regs):
            rs, ss = regs[:K], regs[K:]
            base = cur + b * BLOCK_ELEMS
            vs = [
                row_v[pl.ds(base + j * LANES, LANES)]
                for j in range(BLOCK_VREGS)
            ]
            hs = vs
            ls = vs
            while len(hs) > 1:
                hs = [
                    jnp.maximum(hs[i], hs[i + 1]) for i in range(0, len(hs), 2)
                ]
                ls = [
                    jnp.minimum(ls[i], ls[i + 1]) for i in range(0, len(ls), 2)
                ]
            bm = hs[0]
            bn = ls[0]
            bm_v[pl.ds(b * LANES, LANES)] = bm
            bn_v[pl.ds(b * LANES, LANES)] = bn
            return _insert_desc(rs, bm) + _insert_asc(ss, bn)

        theta_t = _merge_tree(regs[:K], True)[K - 1][0]
        theta_b = _merge_tree(regs[K:], False)[K - 1][0]

        for i in range(K):
            st_v[pl.ds(i * LANES, LANES)] = neg
            st_v[pl.ds((K + i) * LANES, LANES)] = pos

        theta_t_v = jnp.full((LANES,), 1.0, jnp.float32) * theta_t
        theta_b_v = jnp.full((LANES,), 1.0, jnp.float32) * theta_b
        zero = jnp.zeros((LANES,), jnp.float32)
        st_v[pl.ds(2 * K * LANES, LANES)] = zero

        def rescan(b, top):
            cap1 = neg if top else pos
            cap2 = neg if top else pos
            cnt = zero
            for j in range(BLOCK_VREGS):
                v = row_v[pl.ds(cur + b * BLOCK_ELEMS + j * LANES, LANES)]
                if top:
                    m = v >= theta_t_v
                    w = jnp.where(m, v, neg)
                    hi = jnp.maximum(cap1, w)
                    lo = jnp.minimum(cap1, w)
                    cap1 = hi
                    cap2 = jnp.maximum(cap2, lo)
                else:
                    m = v <= theta_b_v
                    w = jnp.where(m, v, pos)
                    lo = jnp.minimum(cap1, w)
                    hi = jnp.maximum(cap1, w)
                    cap1 = lo
                    cap2 = jnp.minimum(cap2, hi)
                cnt = cnt + jnp.where(m, 1.0, 0.0)
            g = st_v[pl.ds(2 * K * LANES, LANES)]
            st_v[pl.ds(2 * K * LANES, LANES)] = jnp.maximum(g, cnt)
            off = 0 if top else K * LANES
            regs = tuple(
                st_v[pl.ds(off + i * LANES, LANES)] for i in range(K)
            )
            if top:
                regs = _insert_desc(_insert_desc(regs, cap1), cap2)
            else:
                regs = _insert_asc(_insert_asc(regs, cap1), cap2)
            for i in range(K):
                st_v[pl.ds(off + i * LANES, LANES)] = regs[i]

        CGROUP = 4

        def c_group(gi, c):
            b0 = gi * CGROUP
            bms = [
                bm_v[pl.ds((b0 + q) * LANES, LANES)] for q in range(CGROUP)
            ]
            bns = [
                bn_v[pl.ds((b0 + q) * LANES, LANES)] for q in range(CGROUP)
            ]
            tts = [bm - theta_t_v for bm in bms]
            tbs = [theta_b_v - bn for bn in bns]
            scomb = [
                _bfly_max(jnp.maximum(tt, tb)) for tt, tb in zip(tts, tbs)
            ]
            for q in range(CGROUP):
                @pl.when(scomb[q][0] >= 0.0)
                def _(q=q):
                    st = _bfly_max(tts[q])
                    sb = _bfly_max(tbs[q])

                    @pl.when(st[0] >= 0.0)
                    def _(b=b0 + q):
                        rescan(b, True)

                    @pl.when(sb[0] >= 0.0)
                    def _(b=b0 + q):
                        rescan(b, False)

            return c

        lax.fori_loop(0, N_BLOCKS // CGROUP, c_group, jnp.int32(0))

        g = st_v[pl.ds(2 * K * LANES, LANES)]
        s_g = _bfly_max(g)[0]

        @pl.when(s_g >= 3.0)
        def _():
            for i in range(K):
                st_v[pl.ds(i * LANES, LANES)] = neg
                st_v[pl.ds((K + i) * LANES, LANES)] = pos

            def c_body(b, c):
                bm = bm_v[pl.ds(b * LANES, LANES)]
                bn = bn_v[pl.ds(b * LANES, LANES)]
                s_mx = _bfly_max(bm)[0]
                s_mn = _bfly_min(bn)[0]

                @pl.when(s_mx >= theta_t)
                def _():
                    rs = tuple(
                        st_v[pl.ds(i * LANES, LANES)] for i in range(K)
                    )
                    for j in range(BLOCK_VREGS):
                        v = row_v[
                            pl.ds(cur + b * BLOCK_ELEMS + j * LANES, LANES)
                        ]
                        rs = _insert_desc(rs, v)
                    for i in range(K):
                        st_v[pl.ds(i * LANES, LANES)] = rs[i]

                @pl.when(s_mn <= theta_b)
                def _():
                    ss = tuple(
                        st_v[pl.ds((K + i) * LANES, LANES)] for i in range(K)
                    )
                    for j in range(BLOCK_VREGS):
                        v = row_v[
                            pl.ds(cur + b * BLOCK_ELEMS + j * LANES, LANES)
                        ]
                        ss = _insert_asc(ss, v)
                    for i in range(K):
                        st_v[pl.ds((K + i) * LANES, LANES)] = ss[i]

                return c

            lax.fori_loop(0, N_BLOCKS, c_body, jnp.int32(0))

        @pl.when(jnp.logical_and(even, t + 2 < ROWS_PER_WORKER))
        def _():
            pltpu.async_copy(x_hbm.at[row + 2], buf(0), sem0)

        @pl.when(jnp.logical_and(jnp.logical_not(even),
                                 t + 2 < ROWS_PER_WORKER))
        def _():
            pltpu.async_copy(x_hbm.at[row + 2], buf(1), sem1)

        rs = tuple(st_v[pl.ds(i * LANES, LANES)] for i in range(K))
        ss = tuple(st_v[pl.ds((K + i) * LANES, LANES)] for i in range(K))
        out_v[pl.ds(t * 2 * LANES, LANES)] = _assemble(_merge_tree(rs, True))
        out_v[pl.ds(t * 2 * LANES + LANES, LANES)] = _assemble(
            _merge_tree(ss, False)
        )
        return carry

    lax.fori_loop(0, ROWS_PER_WORKER, row_work, jnp.int32(0))

    out_len = ROWS_PER_WORKER * 2 * LANES
    pltpu.sync_copy(out_v, out_hbm.at[pl.ds(wid * out_len, out_len)])


@functools.cache
def _get_sc_extreme():
    return pl.kernel(
        _body,
        out_type=jax.ShapeDtypeStruct((N_ROWS * 2 * LANES,), jnp.float32),
        mesh=plsc.VectorSubcoreMesh(
            core_axis_name="c",
            subcore_axis_name="s",
            num_cores=N_CORES,
            num_subcores=N_SUBCORES,
        ),
        scratch_types=[
            pltpu.VMEM((2 * ROW_LEN,), jnp.float32),
            pltpu.VMEM((N_BLOCKS * LANES,), jnp.float32),
            pltpu.VMEM((N_BLOCKS * LANES,), jnp.float32),
            pltpu.VMEM(((2 * K + 1) * LANES,), jnp.float32),
            pltpu.VMEM((ROWS_PER_WORKER * 2 * LANES,), jnp.float32),
            pltpu.SemaphoreType.DMA,
            pltpu.SemaphoreType.DMA,
        ],
    )


@jax.jit
def kernel(x):
    padded = _get_sc_extreme()(x).reshape(N_ROWS, 2 * LANES)
    return jnp.concatenate([padded[:, :K], padded[:, LANES:LANES + K]], axis=1)


# speedup vs baseline: 1.0975x; 1.0149x over previous
import functools

import jax
import jax.numpy as jnp
from jax import lax
from jax.experimental import pallas as pl
from jax.experimental.pallas import tpu as pltpu
from jax.experimental.pallas import tpu_sc as plsc

N_ROWS = 128
ROW_LEN = 32768
K = 10
LANES = 16
BLOCK_VREGS = 16
BLOCK_ELEMS = BLOCK_VREGS * LANES
N_BLOCKS = ROW_LEN // BLOCK_ELEMS

N_CORES = 2
N_SUBCORES = 16
ROWS_PER_WORKER = N_ROWS // (N_CORES * N_SUBCORES)

_NEG = float(-jnp.inf)
_POS = float(jnp.inf)


def _iota():
    return lax.iota(jnp.int32, LANES)


def _insert_desc(regs, v):
    out = []
    c = v
    for r in regs:
        out.append(jnp.maximum(r, c))
        c = jnp.minimum(r, c)
    return tuple(out)


def _insert_asc(regs, v):
    out = []
    c = v
    for r in regs:
        out.append(jnp.minimum(r, c))
        c = jnp.maximum(r, c)
    return tuple(out)


def _bitonic_16(regs, desc):
    regs = list(regs)
    for d in (8, 4, 2, 1):
        for k in range(16):
            if k & d:
                continue
            hi = jnp.maximum(regs[k], regs[k + d])
            lo = jnp.minimum(regs[k], regs[k + d])
            regs[k] = hi if desc else lo
            regs[k + d] = lo if desc else hi
    return regs


def _merge_tree(regs, desc):
    regs = list(regs)
    for dist in (1, 2, 4, 8):
        idx = _iota() ^ dist
        partner = [r[idx] for r in regs]
        n = len(regs)
        merged = []
        for k in range(16):
            a = regs[k] if k < n else None
            b = partner[15 - k] if 15 - k < n else None
            if a is None:
                merged.append(b)
            elif b is None:
                merged.append(a)
            else:
                merged.append(jnp.maximum(a, b) if desc else jnp.minimum(a, b))
        regs = _bitonic_16(merged, desc)
    return regs


def _bfly_max(v):
    for d in (1, 2, 4, 8):
        v = jnp.maximum(v, v[_iota() ^ d])
    return v


def _bfly_min(v):
    for d in (1, 2, 4, 8):
        v = jnp.minimum(v, v[_iota() ^ d])
    return v


def _assemble(regs):
    iota = _iota()
    acc = regs[0]
    for k in range(1, K):
        acc = jnp.where(iota == k, regs[k], acc)
    return acc


def _body(x_hbm, out_hbm, row_v, bm_v, bn_v, st_v, out_v, sem0, sem1):
    wid = lax.axis_index("s") * N_CORES + lax.axis_index("c")

    neg = jnp.full((LANES,), _NEG, jnp.float32)
    pos = jnp.full((LANES,), _POS, jnp.float32)

    row0 = wid * ROWS_PER_WORKER

    def buf(parity):
        return row_v.at[pl.ds(parity * ROW_LEN, ROW_LEN)]

    pltpu.async_copy(x_hbm.at[row0], buf(0), sem0)
    pltpu.async_copy(x_hbm.at[row0 + 1], buf(1), sem1)

    def row_work(t, carry):
        row = row0 + t
        even = t % 2 == 0

        @pl.when(even)
        def _():
            pltpu.make_async_copy(x_hbm.at[row], buf(0), sem0).wait()

        @pl.when(jnp.logical_not(even))
        def _():
            pltpu.make_async_copy(x_hbm.at[row], buf(1), sem1).wait()

        cur = (t % 2) * ROW_LEN

        @plsc.parallel_loop(
            0, N_BLOCKS, unroll=2, carry=(neg,) * K + (pos,) * K
        )
        def regs(b, regs):
            rs, ss = regs[:K], regs[K:]
            base = cur + b * BLOCK_ELEMS
            vs = [
                row_v[pl.ds(base + j * LANES, LANES)]
                for j in range(BLOCK_VREGS)
            ]
            bm = vs[0]
            bn = vs[0]
            for v in vs[1:]:
                bm = jnp.maximum(bm, v)
                bn = jnp.minimum(bn, v)
            bm_v[pl.ds(b * LANES, LANES)] = bm
            bn_v[pl.ds(b * LANES, LANES)] = bn
            return _insert_desc(rs, bm) + _insert_asc(ss, bn)

        theta_t = _merge_tree(regs[:K], True)[K - 1][0]
        theta_b = _merge_tree(regs[K:], False)[K - 1][0]

        for i in range(K):
            st_v[pl.ds(i * LANES, LANES)] = neg
            st_v[pl.ds((K + i) * LANES, LANES)] = pos

        theta_t_v = jnp.full((LANES,), 1.0, jnp.float32) * theta_t
        theta_b_v = jnp.full((LANES,), 1.0, jnp.float32) * theta_b
        zero = jnp.zeros((LANES,), jnp.float32)
        st_v[pl.ds(2 * K * LANES, LANES)] = zero

        def rescan(b, top):
            cap1 = neg if top else pos
            cap2 = neg if top else pos
            cnt = zero
            for j in range(BLOCK_VREGS):
                v = row_v[pl.ds(cur + b * BLOCK_ELEMS + j * LANES, LANES)]
                if top:
                    m = v >= theta_t_v
                    w = jnp.where(m, v, neg)
                    hi = jnp.maximum(cap1, w)
                    lo = jnp.minimum(cap1, w)
                    cap1 = hi
                    cap2 = jnp.maximum(cap2, lo)
                else:
                    m = v <= theta_b_v
                    w = jnp.where(m, v, pos)
                    lo = jnp.minimum(cap1, w)
                    hi = jnp.maximum(cap1, w)
                    cap1 = lo
                    cap2 = jnp.minimum(cap2, hi)
                cnt = cnt + jnp.where(m, 1.0, 0.0)
            g = st_v[pl.ds(2 * K * LANES, LANES)]
            st_v[pl.ds(2 * K * LANES, LANES)] = jnp.maximum(g, cnt)
            off = 0 if top else K * LANES
            regs = tuple(
                st_v[pl.ds(off + i * LANES, LANES)] for i in range(K)
            )
            if top:
                regs = _insert_desc(_insert_desc(regs, cap1), cap2)
            else:
                regs = _insert_asc(_insert_asc(regs, cap1), cap2)
            for i in range(K):
                st_v[pl.ds(off + i * LANES, LANES)] = regs[i]

        CGROUP = 4

        def c_group(gi, c):
            b0 = gi * CGROUP
            bms = [
                bm_v[pl.ds((b0 + q) * LANES, LANES)] for q in range(CGROUP)
            ]
            bns = [
                bn_v[pl.ds((b0 + q) * LANES, LANES)] for q in range(CGROUP)
            ]
            tts = [bm - theta_t_v for bm in bms]
            tbs = [theta_b_v - bn for bn in bns]
            scomb = [
                _bfly_max(jnp.maximum(tt, tb)) for tt, tb in zip(tts, tbs)
            ]
            for q in range(CGROUP):
                @pl.when(scomb[q][0] >= 0.0)
                def _(q=q):
                    st = _bfly_max(tts[q])
                    sb = _bfly_max(tbs[q])

                    @pl.when(st[0] >= 0.0)
                    def _(b=b0 + q):
                        rescan(b, True)

                    @pl.when(sb[0] >= 0.0)
                    def _(b=b0 + q):
                        rescan(b, False)

            return c

        lax.fori_loop(0, N_BLOCKS // CGROUP, c_group, jnp.int32(0))

        g = st_v[pl.ds(2 * K * LANES, LANES)]
        s_g = _bfly_max(g)[0]

        @pl.when(s_g >= 3.0)
        def _():
            for i in range(K):
                st_v[pl.ds(i * LANES, LANES)] = neg
                st_v[pl.ds((K + i) * LANES, LANES)] = pos

            def c_body(b, c):
                bm = bm_v[pl.ds(b * LANES, LANES)]
                bn = bn_v[pl.ds(b * LANES, LANES)]
                s_mx = _bfly_max(bm)[0]
                s_mn = _bfly_min(bn)[0]

                @pl.when(s_mx >= theta_t)
                def _():
                    rs = tuple(
                        st_v[pl.ds(i * LANES, LANES)] for i in range(K)
                    )
                    for j in range(BLOCK_VREGS):
                        v = row_v[
                            pl.ds(cur + b * BLOCK_ELEMS + j * LANES, LANES)
                        ]
                        rs = _insert_desc(rs, v)
                    for i in range(K):
                        st_v[pl.ds(i * LANES, LANES)] = rs[i]

                @pl.when(s_mn <= theta_b)
                def _():
                    ss = tuple(
                        st_v[pl.ds((K + i) * LANES, LANES)] for i in range(K)
                    )
                    for j in range(BLOCK_VREGS):
                        v = row_v[
                            pl.ds(cur + b * BLOCK_ELEMS + j * LANES, LANES)
                        ]
                        ss = _insert_asc(ss, v)
                    for i in range(K):
                        st_v[pl.ds((K + i) * LANES, LANES)] = ss[i]

                return c

            lax.fori_loop(0, N_BLOCKS, c_body, jnp.int32(0))

        @pl.when(jnp.logical_and(even, t + 2 < ROWS_PER_WORKER))
        def _():
            pltpu.async_copy(x_hbm.at[row + 2], buf(0), sem0)

        @pl.when(jnp.logical_and(jnp.logical_not(even),
                                 t + 2 < ROWS_PER_WORKER))
        def _():
            pltpu.async_copy(x_hbm.at[row + 2], buf(1), sem1)

        rs = tuple(st_v[pl.ds(i * LANES, LANES)] for i in range(K))
        ss = tuple(st_v[pl.ds((K + i) * LANES, LANES)] for i in range(K))
        out_v[pl.ds(t * 2 * LANES, LANES)] = _assemble(_merge_tree(rs, True))
        out_v[pl.ds(t * 2 * LANES + LANES, LANES)] = _assemble(
            _merge_tree(ss, False)
        )
        return carry

    lax.fori_loop(0, ROWS_PER_WORKER, row_work, jnp.int32(0))

    out_len = ROWS_PER_WORKER * 2 * LANES
    pltpu.sync_copy(out_v, out_hbm.at[pl.ds(wid * out_len, out_len)])


@functools.cache
def _get_sc_extreme():
    return pl.kernel(
        _body,
        out_type=jax.ShapeDtypeStruct((N_ROWS * 2 * LANES,), jnp.float32),
        mesh=plsc.VectorSubcoreMesh(
            core_axis_name="c",
            subcore_axis_name="s",
            num_cores=N_CORES,
            num_subcores=N_SUBCORES,
        ),
        scratch_types=[
            pltpu.VMEM((2 * ROW_LEN,), jnp.float32),
            pltpu.VMEM((N_BLOCKS * LANES,), jnp.float32),
            pltpu.VMEM((N_BLOCKS * LANES,), jnp.float32),
            pltpu.VMEM(((2 * K + 1) * LANES,), jnp.float32),
            pltpu.VMEM((ROWS_PER_WORKER * 2 * LANES,), jnp.float32),
            pltpu.SemaphoreType.DMA,
            pltpu.SemaphoreType.DMA,
        ],
    )


@jax.jit
def kernel(x):
    padded = _get_sc_extreme()(x).reshape(N_ROWS, 2 * LANES)
    return jnp.concatenate([padded[:, :K], padded[:, LANES:LANES + K]], axis=1)
